# Initial kernel scaffold; baseline (speedup 1.0000x reference)
import jax
import jax.numpy as jnp
from jax.experimental import pallas as pl


def kernel(x, edge_index, edge_weights, W1, b1, W2, b2):
    raise NotImplementedError("write your pallas kernel here")


# speedup vs baseline: 2.6025x; 2.6025x over previous
import jax
import jax.numpy as jnp
from jax import lax
from jax.experimental import pallas as pl
from jax.experimental.pallas import tpu as pltpu
from jax.experimental.pallas import tpu_sc as plsc

_N = 10000
_E = 320000
_D = 128

_NC = 2
_NS = 16
_NT = _NC * _NS
_K = 128
_C = 79
_EPT = _C * _K
_EPAD = _NT * _EPT
_NPAD = 10240
_RPT = _NPAD // _NS
_SB = 14
_SM = (1 << _SB) - 1


def _sc_mesh():
    return plsc.VectorSubcoreMesh(core_axis_name="c", subcore_axis_name="s")


_SC_PARAMS = pltpu.CompilerParams(needs_layout_passes=False)


def _deg_body(pk_hbm, ew_hbm, deg_out, pk_v, ew_v, deg_v):
    cid = lax.axis_index("c")
    sid = lax.axis_index("s")
    wid = sid * _NC + cid
    pltpu.sync_copy(pk_hbm.at[wid], pk_v)
    pltpu.sync_copy(ew_hbm.at[wid], ew_v)

    def _zero(i, carry):
        for g in range(_D // 16):
            deg_v[i, pl.ds(g * 16, 16)] = jnp.zeros((16,), jnp.float32)
        return carry

    lax.fori_loop(0, _NPAD // _D, _zero, 0)

    def _chunk(j, carry):
        for g in range(_K // 16):
            dst16 = lax.shift_right_logical(pk_v[j, pl.ds(g * 16, 16)], _SB)
            w = ew_v[j, pl.ds(g * 16, 16)]
            plsc.addupdate_scatter(
                deg_v,
                [lax.shift_right_logical(dst16, 7),
                 lax.bitwise_and(dst16, _D - 1)],
                w)
        return carry

    lax.fori_loop(0, _C, _chunk, 0)
    pltpu.sync_copy(deg_v, deg_out.at[wid])


def _deg_call(pk, ew):
    return pl.kernel(
        _deg_body,
        out_type=jax.ShapeDtypeStruct((_NT, _NPAD // _D, _D), jnp.float32),
        mesh=_sc_mesh(),
        scratch_types=[
            pltpu.VMEM((_C, _K), jnp.int32),
            pltpu.VMEM((_C, _K), jnp.float32),
            pltpu.VMEM((_NPAD // _D, _D), jnp.float32),
        ],
        compiler_params=_SC_PARAMS,
    )(pk, ew)


def _agg_body(h_hbm, pk_hbm, ew_hbm, q_out, src_v, dst_v, ew_v, rows, qacc,
              gsem):
    cid = lax.axis_index("c")
    sid = lax.axis_index("s")
    wid = sid * _NC + cid
    pltpu.sync_copy(pk_hbm.at[wid], src_v)
    pltpu.sync_copy(ew_hbm.at[wid], ew_v)

    def _unpack(j, carry):
        for g in range(_K // 16):
            p16 = src_v[j, pl.ds(g * 16, 16)]
            dst_v[j, pl.ds(g * 16, 16)] = lax.shift_right_logical(p16, _SB)
            src_v[j, pl.ds(g * 16, 16)] = lax.bitwise_and(p16, _SM)
        return carry

    lax.fori_loop(0, _C, _unpack, 0)

    def _zb(i, carry):
        for g in range(_D // 16):
            rows[i, pl.ds(g * 16, 16)] = jnp.zeros((16,), jnp.float32)
        return carry

    lax.fori_loop(0, _K, _zb, 0)
    for r in range(_RPT // _K):
        pltpu.sync_copy(rows, qacc.at[pl.ds(sid * _RPT + r * _K, _K), :])
    plsc.subcore_barrier()

    lanes = lax.iota(jnp.int32, 16)

    def _chunk(j, carry):
        pltpu.async_copy(h_hbm.at[src_v.at[j]], rows, gsem).wait()

        def _grp(g, c2):
            ew16 = ew_v[j, pl.ds(g * 16, 16)]
            ridx = g * 16 + lanes
            for col in range(_D):
                cidx = jnp.full((16,), col, jnp.int32)
                v = plsc.load_gather(rows, [ridx, cidx])
                plsc.store_scatter(rows, [ridx, cidx], v * ew16)
            return c2

        lax.fori_loop(0, _K // 16, _grp, 0)
        pltpu.sync_copy(rows, qacc.at[dst_v.at[j]], add=True)
        return carry

    lax.fori_loop(0, _C, _chunk, 0)
    plsc.subcore_barrier()
    @pl.when(sid < _NS - 1)
    def _copy_full():
        pltpu.sync_copy(qacc.at[pl.ds(sid * _RPT, _RPT), :],
                        q_out.at[cid, pl.ds(sid * _RPT, _RPT), :])

    @pl.when(sid == _NS - 1)
    def _copy_tail():
        tail = _N - (_NS - 1) * _RPT
        pltpu.sync_copy(qacc.at[pl.ds((_NS - 1) * _RPT, tail), :],
                        q_out.at[cid, pl.ds((_NS - 1) * _RPT, tail), :])


def _agg_call(h, pk, ew):
    return pl.kernel(
        _agg_body,
        out_type=jax.ShapeDtypeStruct((_NC, _N, _D), jnp.float32),
        mesh=_sc_mesh(),
        scratch_types=[
            pltpu.VMEM((_C, _K), jnp.int32),
            pltpu.VMEM((_C, _K), jnp.int32),
            pltpu.VMEM((_C, _K), jnp.float32),
            pltpu.VMEM((_K, _D), jnp.float32),
            pltpu.VMEM_SHARED((_NPAD, _D), jnp.float32),
            pltpu.SemaphoreType.DMA,
        ],
        compiler_params=_SC_PARAMS,
    )(h, pk, ew)


_BN = 1000


def _mm_scale_body(degp_ref, x_ref, w_ref, h_ref, dis_ref):
    deg = 1.0 + jnp.sum(degp_ref[...], axis=1)
    dis = lax.rsqrt(deg)[:, None]
    h_ref[...] = jnp.dot(x_ref[...], w_ref[...],
                         preferred_element_type=jnp.float32) * dis
    dis_ref[...] = dis


def _mm_scale_call(degp, x, w):
    return pl.pallas_call(
        _mm_scale_body,
        grid=(_N // _BN,),
        in_specs=[
            pl.BlockSpec((_BN, _NT), lambda i: (i, 0)),
            pl.BlockSpec((_BN, _D), lambda i: (i, 0)),
            pl.BlockSpec((_D, _D), lambda i: (0, 0)),
        ],
        out_specs=[
            pl.BlockSpec((_BN, _D), lambda i: (i, 0)),
            pl.BlockSpec((_BN, 1), lambda i: (i, 0)),
        ],
        out_shape=[
            jax.ShapeDtypeStruct((_N, _D), jnp.float32),
            jax.ShapeDtypeStruct((_N, 1), jnp.float32),
        ],
    )(degp, x, w)


def _mid_body(q_ref, hp_ref, dis_ref, b1_ref, w2_ref, out_ref):
    dis = dis_ref[...]
    t = (q_ref[0] + q_ref[1] + hp_ref[...]) * dis + b1_ref[...]
    t = jnp.maximum(t, 0.0)
    out_ref[...] = jnp.dot(t, w2_ref[...],
                           preferred_element_type=jnp.float32) * dis


def _mid_call(q, hp, dis, b1, w2):
    return pl.pallas_call(
        _mid_body,
        grid=(_N // _BN,),
        in_specs=[
            pl.BlockSpec((_NC, _BN, _D), lambda i: (0, i, 0)),
            pl.BlockSpec((_BN, _D), lambda i: (i, 0)),
            pl.BlockSpec((_BN, 1), lambda i: (i, 0)),
            pl.BlockSpec((1, _D), lambda i: (0, 0)),
            pl.BlockSpec((_D, _D), lambda i: (0, 0)),
        ],
        out_specs=pl.BlockSpec((_BN, _D), lambda i: (i, 0)),
        out_shape=jax.ShapeDtypeStruct((_N, _D), jnp.float32),
    )(q, hp, dis, b1, w2)


def _fin_body(q_ref, hp_ref, dis_ref, b2_ref, out_ref):
    out_ref[...] = ((q_ref[0] + q_ref[1] + hp_ref[...]) * dis_ref[...]
                    + b2_ref[...])


def _fin_call(q, hp, dis, b2):
    return pl.pallas_call(
        _fin_body,
        grid=(_N // _BN,),
        in_specs=[
            pl.BlockSpec((_NC, _BN, _D), lambda i: (0, i, 0)),
            pl.BlockSpec((_BN, _D), lambda i: (i, 0)),
            pl.BlockSpec((_BN, 1), lambda i: (i, 0)),
            pl.BlockSpec((1, _D), lambda i: (0, 0)),
        ],
        out_specs=pl.BlockSpec((_BN, _D), lambda i: (i, 0)),
        out_shape=jax.ShapeDtypeStruct((_N, _D), jnp.float32),
    )(q, hp, dis, b2)


def kernel(x, edge_index, edge_weights, W1, b1, W2, b2):
    src = edge_index[0]
    dst = edge_index[1]
    pad = _EPAD - _E
    pk = jnp.concatenate(
        [src + (dst << _SB), jnp.zeros((pad,), jnp.int32)]
    ).reshape(_NT, _C, _K)
    ew = jnp.concatenate(
        [edge_weights, jnp.zeros((pad,), jnp.float32)]
    ).reshape(_NT, _C, _K)

    degp = _deg_call(pk, ew).reshape(_NT, _NPAD)[:, :_N].T
    h1p, dis = _mm_scale_call(degp, x, W1)
    q1 = _agg_call(h1p, pk, ew)
    h2p = _mid_call(q1, h1p, dis, b1.reshape(1, _D), W2)
    q2 = _agg_call(h2p, pk, ew)
    out = _fin_call(q2, h2p, dis, b2.reshape(1, _D))
    return out


# speedup vs baseline: 2.9433x; 1.1309x over previous
import jax
import jax.numpy as jnp
from jax import lax
from jax.experimental import pallas as pl
from jax.experimental.pallas import tpu as pltpu
from jax.experimental.pallas import tpu_sc as plsc

_N = 10000
_E = 320000
_D = 128

_NC = 2
_NS = 16
_NT = _NC * _NS
_K = 128
_C = 80
_EPT = _C * _K
_EPAD = _NT * _EPT
_NPAD = 10240
_RPT = _NPAD // _NS
_SB = 14
_SM = (1 << _SB) - 1


def _sc_mesh():
    return plsc.VectorSubcoreMesh(core_axis_name="c", subcore_axis_name="s")


_SC_PARAMS = pltpu.CompilerParams(needs_layout_passes=False)


def _deg_body(pk_hbm, ew_hbm, deg_out, pk_v, ew_v, deg_v):
    cid = lax.axis_index("c")
    sid = lax.axis_index("s")
    wid = sid * _NC + cid
    pltpu.sync_copy(pk_hbm.at[wid], pk_v)
    pltpu.sync_copy(ew_hbm.at[wid], ew_v)

    def _zero(i, carry):
        for g in range(_D // 16):
            deg_v[i, pl.ds(g * 16, 16)] = jnp.zeros((16,), jnp.float32)
        return carry

    lax.fori_loop(0, _NPAD // _D, _zero, 0)

    def _chunk(j, carry):
        for g in range(_K // 16):
            dst16 = lax.shift_right_logical(pk_v[j, pl.ds(g * 16, 16)], _SB)
            w = ew_v[j, pl.ds(g * 16, 16)]
            plsc.addupdate_scatter(
                deg_v,
                [lax.shift_right_logical(dst16, 7),
                 lax.bitwise_and(dst16, _D - 1)],
                w)
        return carry

    lax.fori_loop(0, _C, _chunk, 0)
    pltpu.sync_copy(deg_v, deg_out.at[wid])


def _deg_call(pk, ew):
    return pl.kernel(
        _deg_body,
        out_type=jax.ShapeDtypeStruct((_NT, _NPAD // _D, _D), jnp.float32),
        mesh=_sc_mesh(),
        scratch_types=[
            pltpu.VMEM((_C, _K), jnp.int32),
            pltpu.VMEM((_C, _K), jnp.float32),
            pltpu.VMEM((_NPAD // _D, _D), jnp.float32),
        ],
        compiler_params=_SC_PARAMS,
    )(pk, ew)


def _agg_body(h_hbm, pk_hbm, ew_hbm, q_out, pk_v, src_ch, dst_ch,
              ew_a, ew_b, rows_a, rows_b, qacc,
              gsem_a, gsem_b, esem_a, esem_b):
    cid = lax.axis_index("c")
    sid = lax.axis_index("s")
    wid = sid * _NC + cid
    pltpu.sync_copy(pk_hbm.at[wid], pk_v)

    def _zb(i, carry):
        for g in range(_D // 16):
            rows_a[i, pl.ds(g * 16, 16)] = jnp.zeros((16,), jnp.float32)
        return carry

    lax.fori_loop(0, _K, _zb, 0)
    for r in range(_RPT // _K):
        pltpu.sync_copy(rows_a, qacc.at[pl.ds(sid * _RPT + r * _K, _K), :])
    plsc.subcore_barrier()

    lanes = lax.iota(jnp.int32, 16)

    def _unpack(j, q):
        for g in range(_K // 16):
            p16 = pk_v[j, pl.ds(g * 16, 16)]
            dst_ch[q, pl.ds(g * 16, 16)] = lax.shift_right_logical(p16, _SB)
            src_ch[q, pl.ds(g * 16, 16)] = lax.bitwise_and(p16, _SM)

    def _fire(j, q, rows, sem):
        _unpack(j, q)
        pltpu.make_async_copy(h_hbm.at[src_ch.at[q]], rows, sem).start()

    def _wait_rows(q, rows, sem):
        pltpu.make_async_copy(h_hbm.at[src_ch.at[q]], rows, sem).wait()

    def _scale(ewbuf, er, rows):
        def _grp(g, c2):
            ew16 = ewbuf[er, pl.ds(g * 16, 16)]
            ridx = g * 16 + lanes

            def _col(c0, c3):
                cidx = jnp.full((16,), c0, jnp.int32)
                v = plsc.load_gather(rows, [ridx, cidx])
                plsc.store_scatter(rows, [ridx, cidx], v * ew16)
                return c3

            lax.fori_loop(0, _D, _col, 0, unroll=16)
            return c2

        lax.fori_loop(0, _K // 16, _grp, 0)

    def _ew_copy(s, buf, sem):
        return pltpu.make_async_copy(ew_hbm.at[wid, pl.ds(s * 8, 8)],
                                     buf, sem)

    _fire(0, 0, rows_a, gsem_a)
    _ew_copy(0, ew_a, esem_a).start()

    def _super_pair(sp, carry):
        for half in (0, 1):
            base = sp * 16 + half * 8
            if half == 0:
                _ew_copy(2 * sp, ew_a, esem_a).wait()
                _ew_copy(2 * sp + 1, ew_b, esem_b).start()
                ewbuf = ew_a
            else:
                _ew_copy(2 * sp + 1, ew_b, esem_b).wait()

                @pl.when(sp < 4)
                def _pf():
                    _ew_copy(2 * sp + 2, ew_a, esem_a).start()

                ewbuf = ew_b

            def _pair(pr, c2, base=base, ewbuf=ewbuf):
                a = base + 2 * pr
                _fire(a + 1, 1, rows_b, gsem_b)
                _wait_rows(0, rows_a, gsem_a)
                _scale(ewbuf, 2 * pr, rows_a)
                pltpu.sync_copy(rows_a, qacc.at[dst_ch.at[0]], add=True)

                @pl.when(a + 2 < _C)
                def _fn():
                    _fire(a + 2, 0, rows_a, gsem_a)

                _wait_rows(1, rows_b, gsem_b)
                _scale(ewbuf, 2 * pr + 1, rows_b)
                pltpu.sync_copy(rows_b, qacc.at[dst_ch.at[1]], add=True)
                return c2

            lax.fori_loop(0, 4, _pair, 0)
        return carry

    lax.fori_loop(0, _C // 16, _super_pair, 0)
    plsc.subcore_barrier()
    @pl.when(sid < _NS - 1)
    def _copy_full():
        pltpu.sync_copy(qacc.at[pl.ds(sid * _RPT, _RPT), :],
                        q_out.at[cid, pl.ds(sid * _RPT, _RPT), :])

    @pl.when(sid == _NS - 1)
    def _copy_tail():
        tail = _N - (_NS - 1) * _RPT
        pltpu.sync_copy(qacc.at[pl.ds((_NS - 1) * _RPT, tail), :],
                        q_out.at[cid, pl.ds((_NS - 1) * _RPT, tail), :])


def _agg_call(h, pk, ew):
    return pl.kernel(
        _agg_body,
        out_type=jax.ShapeDtypeStruct((_NC, _N, _D), jnp.float32),
        mesh=_sc_mesh(),
        scratch_types=[
            pltpu.VMEM((_C, _K), jnp.int32),
            pltpu.VMEM((8, _K), jnp.int32),
            pltpu.VMEM((8, _K), jnp.int32),
            pltpu.VMEM((8, _K), jnp.float32),
            pltpu.VMEM((8, _K), jnp.float32),
            pltpu.VMEM((_K, _D), jnp.float32),
            pltpu.VMEM((_K, _D), jnp.float32),
            pltpu.VMEM_SHARED((_NPAD, _D), jnp.float32),
            pltpu.SemaphoreType.DMA,
            pltpu.SemaphoreType.DMA,
            pltpu.SemaphoreType.DMA,
            pltpu.SemaphoreType.DMA,
        ],
        compiler_params=_SC_PARAMS,
    )(h, pk, ew)


_BN = 1000


def _mm_scale_body(degp_ref, x_ref, w_ref, h_ref, dis_ref):
    deg = 1.0 + jnp.sum(degp_ref[...], axis=1)
    dis = lax.rsqrt(deg)[:, None]
    h_ref[...] = jnp.dot(x_ref[...], w_ref[...],
                         preferred_element_type=jnp.float32) * dis
    dis_ref[...] = dis


def _mm_scale_call(degp, x, w):
    return pl.pallas_call(
        _mm_scale_body,
        grid=(_N // _BN,),
        in_specs=[
            pl.BlockSpec((_BN, _NT), lambda i: (i, 0)),
            pl.BlockSpec((_BN, _D), lambda i: (i, 0)),
            pl.BlockSpec((_D, _D), lambda i: (0, 0)),
        ],
        out_specs=[
            pl.BlockSpec((_BN, _D), lambda i: (i, 0)),
            pl.BlockSpec((_BN, 1), lambda i: (i, 0)),
        ],
        out_shape=[
            jax.ShapeDtypeStruct((_N, _D), jnp.float32),
            jax.ShapeDtypeStruct((_N, 1), jnp.float32),
        ],
    )(degp, x, w)


def _mid_body(q_ref, hp_ref, dis_ref, b1_ref, w2_ref, out_ref):
    dis = dis_ref[...]
    t = (q_ref[0] + q_ref[1] + hp_ref[...]) * dis + b1_ref[...]
    t = jnp.maximum(t, 0.0)
    out_ref[...] = jnp.dot(t, w2_ref[...],
                           preferred_element_type=jnp.float32) * dis


def _mid_call(q, hp, dis, b1, w2):
    return pl.pallas_call(
        _mid_body,
        grid=(_N // _BN,),
        in_specs=[
            pl.BlockSpec((_NC, _BN, _D), lambda i: (0, i, 0)),
            pl.BlockSpec((_BN, _D), lambda i: (i, 0)),
            pl.BlockSpec((_BN, 1), lambda i: (i, 0)),
            pl.BlockSpec((1, _D), lambda i: (0, 0)),
            pl.BlockSpec((_D, _D), lambda i: (0, 0)),
        ],
        out_specs=pl.BlockSpec((_BN, _D), lambda i: (i, 0)),
        out_shape=jax.ShapeDtypeStruct((_N, _D), jnp.float32),
    )(q, hp, dis, b1, w2)


def _fin_body(q_ref, hp_ref, dis_ref, b2_ref, out_ref):
    out_ref[...] = ((q_ref[0] + q_ref[1] + hp_ref[...]) * dis_ref[...]
                    + b2_ref[...])


def _fin_call(q, hp, dis, b2):
    return pl.pallas_call(
        _fin_body,
        grid=(_N // _BN,),
        in_specs=[
            pl.BlockSpec((_NC, _BN, _D), lambda i: (0, i, 0)),
            pl.BlockSpec((_BN, _D), lambda i: (i, 0)),
            pl.BlockSpec((_BN, 1), lambda i: (i, 0)),
            pl.BlockSpec((1, _D), lambda i: (0, 0)),
        ],
        out_specs=pl.BlockSpec((_BN, _D), lambda i: (i, 0)),
        out_shape=jax.ShapeDtypeStruct((_N, _D), jnp.float32),
    )(q, hp, dis, b2)


def kernel(x, edge_index, edge_weights, W1, b1, W2, b2):
    src = edge_index[0]
    dst = edge_index[1]
    pad = _EPAD - _E
    pk = jnp.concatenate(
        [src + (dst << _SB), jnp.zeros((pad,), jnp.int32)]
    ).reshape(_NT, _C, _K)
    ew = jnp.concatenate(
        [edge_weights, jnp.zeros((pad,), jnp.float32)]
    ).reshape(_NT, _C, _K)

    degp = _deg_call(pk, ew).reshape(_NT, _NPAD)[:, :_N].T
    h1p, dis = _mm_scale_call(degp, x, W1)
    q1 = _agg_call(h1p, pk, ew)
    h2p = _mid_call(q1, h1p, dis, b1.reshape(1, _D), W2)
    q2 = _agg_call(h2p, pk, ew)
    out = _fin_call(q2, h2p, dis, b2.reshape(1, _D))
    return out


# speedup vs baseline: 5.3153x; 1.8059x over previous
import jax
import jax.numpy as jnp
from jax import lax
from jax.experimental import pallas as pl
from jax.experimental.pallas import tpu as pltpu
from jax.experimental.pallas import tpu_sc as plsc

_N = 10000
_E = 320000
_D = 128

_NC = 2
_NS = 16
_NT = _NC * _NS
_K = 128
_C = 80
_EPT = _C * _K
_EPAD = _NT * _EPT
_NPAD = 10240
_RPT = _NPAD // _NS
_SB = 14
_SM = (1 << _SB) - 1


def _sc_mesh():
    return plsc.VectorSubcoreMesh(core_axis_name="c", subcore_axis_name="s")


_SC_PARAMS = pltpu.CompilerParams(needs_layout_passes=False)


def _deg_body(pk_hbm, ew_hbm, deg_out, pk_v, ew_v, deg_v):
    cid = lax.axis_index("c")
    sid = lax.axis_index("s")
    wid = sid * _NC + cid
    pltpu.sync_copy(pk_hbm.at[wid], pk_v)
    pltpu.sync_copy(ew_hbm.at[wid], ew_v)

    def _zero(i, carry):
        for g in range(_D // 16):
            deg_v[i, pl.ds(g * 16, 16)] = jnp.zeros((16,), jnp.float32)
        return carry

    lax.fori_loop(0, _NPAD // _D, _zero, 0)

    def _chunk(j, carry):
        for g in range(_K // 16):
            dst16 = lax.shift_right_logical(pk_v[j, pl.ds(g * 16, 16)], _SB)
            w = ew_v[j, pl.ds(g * 16, 16)]
            plsc.addupdate_scatter(
                deg_v,
                [lax.shift_right_logical(dst16, 7),
                 lax.bitwise_and(dst16, _D - 1)],
                w)
        return carry

    lax.fori_loop(0, _C, _chunk, 0)
    pltpu.sync_copy(deg_v, deg_out.at[wid])


def _deg_call(pk, ew):
    return pl.kernel(
        _deg_body,
        out_type=jax.ShapeDtypeStruct((_NT, _NPAD // _D, _D), jnp.float32),
        mesh=_sc_mesh(),
        scratch_types=[
            pltpu.VMEM((_C, _K), jnp.int32),
            pltpu.VMEM((_C, _K), jnp.float32),
            pltpu.VMEM((_NPAD // _D, _D), jnp.float32),
        ],
        compiler_params=_SC_PARAMS,
    )(pk, ew)


def _agg_body(h_hbm, pk_hbm, ew_hbm, q_out, pk_v, src_ch, dst_ch,
              ew_a, ew_b, rows_a, rows_b, qacc,
              gsem_a, gsem_b, esem_a, esem_b):
    cid = lax.axis_index("c")
    sid = lax.axis_index("s")
    wid = sid * _NC + cid
    pltpu.sync_copy(pk_hbm.at[wid], pk_v)

    def _zb(i, carry):
        for g in range(_D // 16):
            rows_a[i, pl.ds(g * 16, 16)] = jnp.zeros((16,), jnp.float32)
        return carry

    lax.fori_loop(0, _K, _zb, 0)
    for r in range(_RPT // _K):
        pltpu.sync_copy(rows_a, qacc.at[pl.ds(sid * _RPT + r * _K, _K), :])
    plsc.subcore_barrier()

    lanes = lax.iota(jnp.int32, 16)

    def _unpack(j, q):
        for g in range(_K // 16):
            p16 = pk_v[j, pl.ds(g * 16, 16)]
            dst_ch[q, pl.ds(g * 16, 16)] = lax.shift_right_logical(p16, _SB)
            src_ch[q, pl.ds(g * 16, 16)] = lax.bitwise_and(p16, _SM)

    def _fire(j, q, rows, sem):
        _unpack(j, q)
        pltpu.make_async_copy(h_hbm.at[src_ch.at[q]], rows, sem).start()

    def _wait_rows(q, rows, sem):
        pltpu.make_async_copy(h_hbm.at[src_ch.at[q]], rows, sem).wait()

    def _scale(ewbuf, er, rows):
        def _grp(g, c2):
            ew16 = ewbuf[er, pl.ds(g * 16, 16)]
            ridx = g * 16 + lanes

            @plsc.parallel_loop(0, _D, 1, unroll=16)
            def _col(c0):
                cidx = jnp.full((16,), c0, jnp.int32)
                v = plsc.load_gather(rows, [ridx, cidx])
                plsc.store_scatter(rows, [ridx, cidx], v * ew16)

            return c2

        lax.fori_loop(0, _K // 16, _grp, 0)

    def _ew_copy(s, buf, sem):
        return pltpu.make_async_copy(ew_hbm.at[wid, pl.ds(s * 8, 8)],
                                     buf, sem)

    _fire(0, 0, rows_a, gsem_a)
    _ew_copy(0, ew_a, esem_a).start()

    def _super_pair(sp, carry):
        for half in (0, 1):
            base = sp * 16 + half * 8
            if half == 0:
                _ew_copy(2 * sp, ew_a, esem_a).wait()
                _ew_copy(2 * sp + 1, ew_b, esem_b).start()
                ewbuf = ew_a
            else:
                _ew_copy(2 * sp + 1, ew_b, esem_b).wait()

                @pl.when(sp < 4)
                def _pf():
                    _ew_copy(2 * sp + 2, ew_a, esem_a).start()

                ewbuf = ew_b

            def _pair(pr, c2, base=base, ewbuf=ewbuf):
                a = base + 2 * pr
                _fire(a + 1, 1, rows_b, gsem_b)
                _wait_rows(0, rows_a, gsem_a)
                _scale(ewbuf, 2 * pr, rows_a)
                pltpu.sync_copy(rows_a, qacc.at[dst_ch.at[0]], add=True)

                @pl.when(a + 2 < _C)
                def _fn():
                    _fire(a + 2, 0, rows_a, gsem_a)

                _wait_rows(1, rows_b, gsem_b)
                _scale(ewbuf, 2 * pr + 1, rows_b)
                pltpu.sync_copy(rows_b, qacc.at[dst_ch.at[1]], add=True)
                return c2

            lax.fori_loop(0, 4, _pair, 0)
        return carry

    lax.fori_loop(0, _C // 16, _super_pair, 0)
    plsc.subcore_barrier()
    @pl.when(sid < _NS - 1)
    def _copy_full():
        pltpu.sync_copy(qacc.at[pl.ds(sid * _RPT, _RPT), :],
                        q_out.at[cid, pl.ds(sid * _RPT, _RPT), :])

    @pl.when(sid == _NS - 1)
    def _copy_tail():
        tail = _N - (_NS - 1) * _RPT
        pltpu.sync_copy(qacc.at[pl.ds((_NS - 1) * _RPT, tail), :],
                        q_out.at[cid, pl.ds((_NS - 1) * _RPT, tail), :])


def _agg_call(h, pk, ew):
    return pl.kernel(
        _agg_body,
        out_type=jax.ShapeDtypeStruct((_NC, _N, _D), jnp.float32),
        mesh=_sc_mesh(),
        scratch_types=[
            pltpu.VMEM((_C, _K), jnp.int32),
            pltpu.VMEM((8, _K), jnp.int32),
            pltpu.VMEM((8, _K), jnp.int32),
            pltpu.VMEM((8, _K), jnp.float32),
            pltpu.VMEM((8, _K), jnp.float32),
            pltpu.VMEM((_K, _D), jnp.float32),
            pltpu.VMEM((_K, _D), jnp.float32),
            pltpu.VMEM_SHARED((_NPAD, _D), jnp.float32),
            pltpu.SemaphoreType.DMA,
            pltpu.SemaphoreType.DMA,
            pltpu.SemaphoreType.DMA,
            pltpu.SemaphoreType.DMA,
        ],
        compiler_params=_SC_PARAMS,
    )(h, pk, ew)


_BN = 1000


def _mm_scale_body(degp_ref, x_ref, w_ref, h_ref, dis_ref):
    deg = 1.0 + jnp.sum(degp_ref[...], axis=1)
    dis = lax.rsqrt(deg)[:, None]
    h_ref[...] = jnp.dot(x_ref[...], w_ref[...],
                         preferred_element_type=jnp.float32) * dis
    dis_ref[...] = dis


def _mm_scale_call(degp, x, w):
    return pl.pallas_call(
        _mm_scale_body,
        grid=(_N // _BN,),
        in_specs=[
            pl.BlockSpec((_BN, _NT), lambda i: (i, 0)),
            pl.BlockSpec((_BN, _D), lambda i: (i, 0)),
            pl.BlockSpec((_D, _D), lambda i: (0, 0)),
        ],
        out_specs=[
            pl.BlockSpec((_BN, _D), lambda i: (i, 0)),
            pl.BlockSpec((_BN, 1), lambda i: (i, 0)),
        ],
        out_shape=[
            jax.ShapeDtypeStruct((_N, _D), jnp.float32),
            jax.ShapeDtypeStruct((_N, 1), jnp.float32),
        ],
    )(degp, x, w)


def _mid_body(q_ref, hp_ref, dis_ref, b1_ref, w2_ref, out_ref):
    dis = dis_ref[...]
    t = (q_ref[0] + q_ref[1] + hp_ref[...]) * dis + b1_ref[...]
    t = jnp.maximum(t, 0.0)
    out_ref[...] = jnp.dot(t, w2_ref[...],
                           preferred_element_type=jnp.float32) * dis


def _mid_call(q, hp, dis, b1, w2):
    return pl.pallas_call(
        _mid_body,
        grid=(_N // _BN,),
        in_specs=[
            pl.BlockSpec((_NC, _BN, _D), lambda i: (0, i, 0)),
            pl.BlockSpec((_BN, _D), lambda i: (i, 0)),
            pl.BlockSpec((_BN, 1), lambda i: (i, 0)),
            pl.BlockSpec((1, _D), lambda i: (0, 0)),
            pl.BlockSpec((_D, _D), lambda i: (0, 0)),
        ],
        out_specs=pl.BlockSpec((_BN, _D), lambda i: (i, 0)),
        out_shape=jax.ShapeDtypeStruct((_N, _D), jnp.float32),
    )(q, hp, dis, b1, w2)


def _fin_body(q_ref, hp_ref, dis_ref, b2_ref, out_ref):
    out_ref[...] = ((q_ref[0] + q_ref[1] + hp_ref[...]) * dis_ref[...]
                    + b2_ref[...])


def _fin_call(q, hp, dis, b2):
    return pl.pallas_call(
        _fin_body,
        grid=(_N // _BN,),
        in_specs=[
            pl.BlockSpec((_NC, _BN, _D), lambda i: (0, i, 0)),
            pl.BlockSpec((_BN, _D), lambda i: (i, 0)),
            pl.BlockSpec((_BN, 1), lambda i: (i, 0)),
            pl.BlockSpec((1, _D), lambda i: (0, 0)),
        ],
        out_specs=pl.BlockSpec((_BN, _D), lambda i: (i, 0)),
        out_shape=jax.ShapeDtypeStruct((_N, _D), jnp.float32),
    )(q, hp, dis, b2)


def kernel(x, edge_index, edge_weights, W1, b1, W2, b2):
    src = edge_index[0]
    dst = edge_index[1]
    pad = _EPAD - _E
    pk = jnp.concatenate(
        [src + (dst << _SB), jnp.zeros((pad,), jnp.int32)]
    ).reshape(_NT, _C, _K)
    ew = jnp.concatenate(
        [edge_weights, jnp.zeros((pad,), jnp.float32)]
    ).reshape(_NT, _C, _K)

    degp = _deg_call(pk, ew).reshape(_NT, _NPAD)[:, :_N].T
    h1p, dis = _mm_scale_call(degp, x, W1)
    q1 = _agg_call(h1p, pk, ew)
    h2p = _mid_call(q1, h1p, dis, b1.reshape(1, _D), W2)
    q2 = _agg_call(h2p, pk, ew)
    out = _fin_call(q2, h2p, dis, b2.reshape(1, _D))
    return out


# speedup vs baseline: 9.8967x; 1.8619x over previous
import jax
import jax.numpy as jnp
from jax import lax
from jax.experimental import pallas as pl
from jax.experimental.pallas import tpu as pltpu
from jax.experimental.pallas import tpu_sc as plsc

_N = 10000
_E = 320000
_D = 128

_NC = 2
_NS = 16
_NT = _NC * _NS
_K = 128
_C = 80
_EPT = _C * _K
_EPAD = _NT * _EPT
_NPAD = 10240
_RPT = _NPAD // _NS
_SB = 14
_SM = (1 << _SB) - 1


def _sc_mesh():
    return plsc.VectorSubcoreMesh(core_axis_name="c", subcore_axis_name="s")


_SC_PARAMS = pltpu.CompilerParams(needs_layout_passes=False)


def _deg_body(pk_hbm, ew_hbm, deg_out, pk_v, ew_v, deg_v):
    cid = lax.axis_index("c")
    sid = lax.axis_index("s")
    wid = sid * _NC + cid
    pltpu.sync_copy(pk_hbm.at[wid], pk_v)
    pltpu.sync_copy(ew_hbm.at[wid], ew_v)

    def _zero(i, carry):
        for g in range(_D // 16):
            deg_v[i, pl.ds(g * 16, 16)] = jnp.zeros((16,), jnp.float32)
        return carry

    lax.fori_loop(0, _NPAD // _D, _zero, 0)

    def _chunk(j, carry):
        for g in range(_K // 16):
            dst16 = lax.shift_right_logical(pk_v[j, pl.ds(g * 16, 16)], _SB)
            w = ew_v[j, pl.ds(g * 16, 16)]
            plsc.addupdate_scatter(
                deg_v,
                [lax.shift_right_logical(dst16, 7),
                 lax.bitwise_and(dst16, _D - 1)],
                w)
        return carry

    lax.fori_loop(0, _C, _chunk, 0)
    pltpu.sync_copy(deg_v, deg_out.at[wid])


def _deg_call(pk, ew):
    return pl.kernel(
        _deg_body,
        out_type=jax.ShapeDtypeStruct((_NT, _NPAD // _D, _D), jnp.float32),
        mesh=_sc_mesh(),
        scratch_types=[
            pltpu.VMEM((_C, _K), jnp.int32),
            pltpu.VMEM((_C, _K), jnp.float32),
            pltpu.VMEM((_NPAD // _D, _D), jnp.float32),
        ],
        compiler_params=_SC_PARAMS,
    )(pk, ew)


def _agg_body(h_hbm, pk_hbm, ew_hbm, q_out, pk_v, src_ch, dst_ch,
              ew_a, ew_b, rows_a, rows_b, qacc,
              gsem_a, gsem_b, esem_a, esem_b):
    cid = lax.axis_index("c")
    sid = lax.axis_index("s")
    wid = sid * _NC + cid
    pltpu.sync_copy(pk_hbm.at[wid], pk_v)

    def _zb(i, carry):
        for g in range(_D // 16):
            rows_a[i, pl.ds(g * 16, 16)] = jnp.zeros((16,), jnp.float32)
        return carry

    lax.fori_loop(0, _K, _zb, 0)
    for r in range(_RPT // _K):
        pltpu.sync_copy(rows_a, qacc.at[pl.ds(sid * _RPT + r * _K, _K), :])
    plsc.subcore_barrier()

    lanes = lax.iota(jnp.int32, 16)

    def _unpack(j, q):
        for g in range(_K // 16):
            p16 = pk_v[j, pl.ds(g * 16, 16)]
            dst_ch[q, pl.ds(g * 16, 16)] = lax.shift_right_logical(p16, _SB)
            src_ch[q, pl.ds(g * 16, 16)] = lax.bitwise_and(p16, _SM)

    def _fire(j, q, rows, sem):
        _unpack(j, q)
        pltpu.make_async_copy(h_hbm.at[src_ch.at[q]], rows, sem).start()

    def _wait_rows(q, rows, sem):
        pltpu.make_async_copy(h_hbm.at[src_ch.at[q]], rows, sem).wait()

    def _scale(ewbuf, er, rows):
        def _grp(g, c2):
            ew16 = ewbuf[er, pl.ds(g * 16, 16)]
            ridx = g * 16 + lanes

            @plsc.parallel_loop(0, _D, 1, unroll=16)
            def _col(c0):
                cidx = lax.bitwise_and(c0 + lanes, _D - 1)
                v = plsc.load_gather(rows, [ridx, cidx])
                plsc.store_scatter(rows, [ridx, cidx], v * ew16)

            return c2

        lax.fori_loop(0, _K // 16, _grp, 0)

    def _ew_copy(s, buf, sem):
        return pltpu.make_async_copy(ew_hbm.at[wid, pl.ds(s * 8, 8)],
                                     buf, sem)

    _fire(0, 0, rows_a, gsem_a)
    _ew_copy(0, ew_a, esem_a).start()

    def _super_pair(sp, carry):
        for half in (0, 1):
            base = sp * 16 + half * 8
            if half == 0:
                _ew_copy(2 * sp, ew_a, esem_a).wait()
                _ew_copy(2 * sp + 1, ew_b, esem_b).start()
                ewbuf = ew_a
            else:
                _ew_copy(2 * sp + 1, ew_b, esem_b).wait()

                @pl.when(sp < 4)
                def _pf():
                    _ew_copy(2 * sp + 2, ew_a, esem_a).start()

                ewbuf = ew_b

            def _pair(pr, c2, base=base, ewbuf=ewbuf):
                a = base + 2 * pr
                _fire(a + 1, 1, rows_b, gsem_b)
                _wait_rows(0, rows_a, gsem_a)
                _scale(ewbuf, 2 * pr, rows_a)
                pltpu.sync_copy(rows_a, qacc.at[dst_ch.at[0]], add=True)

                @pl.when(a + 2 < _C)
                def _fn():
                    _fire(a + 2, 0, rows_a, gsem_a)

                _wait_rows(1, rows_b, gsem_b)
                _scale(ewbuf, 2 * pr + 1, rows_b)
                pltpu.sync_copy(rows_b, qacc.at[dst_ch.at[1]], add=True)
                return c2

            lax.fori_loop(0, 4, _pair, 0)
        return carry

    lax.fori_loop(0, _C // 16, _super_pair, 0)
    plsc.subcore_barrier()
    @pl.when(sid < _NS - 1)
    def _copy_full():
        pltpu.sync_copy(qacc.at[pl.ds(sid * _RPT, _RPT), :],
                        q_out.at[cid, pl.ds(sid * _RPT, _RPT), :])

    @pl.when(sid == _NS - 1)
    def _copy_tail():
        tail = _N - (_NS - 1) * _RPT
        pltpu.sync_copy(qacc.at[pl.ds((_NS - 1) * _RPT, tail), :],
                        q_out.at[cid, pl.ds((_NS - 1) * _RPT, tail), :])


def _agg_call(h, pk, ew):
    return pl.kernel(
        _agg_body,
        out_type=jax.ShapeDtypeStruct((_NC, _N, _D), jnp.float32),
        mesh=_sc_mesh(),
        scratch_types=[
            pltpu.VMEM((_C, _K), jnp.int32),
            pltpu.VMEM((8, _K), jnp.int32),
            pltpu.VMEM((8, _K), jnp.int32),
            pltpu.VMEM((8, _K), jnp.float32),
            pltpu.VMEM((8, _K), jnp.float32),
            pltpu.VMEM((_K, _D), jnp.float32),
            pltpu.VMEM((_K, _D), jnp.float32),
            pltpu.VMEM_SHARED((_NPAD, _D), jnp.float32),
            pltpu.SemaphoreType.DMA,
            pltpu.SemaphoreType.DMA,
            pltpu.SemaphoreType.DMA,
            pltpu.SemaphoreType.DMA,
        ],
        compiler_params=_SC_PARAMS,
    )(h, pk, ew)


_BN = 1000


def _mm_scale_body(degp_ref, x_ref, w_ref, h_ref, dis_ref):
    deg = 1.0 + jnp.sum(degp_ref[...], axis=1)
    dis = lax.rsqrt(deg)[:, None]
    h_ref[...] = jnp.dot(x_ref[...], w_ref[...],
                         preferred_element_type=jnp.float32) * dis
    dis_ref[...] = dis


def _mm_scale_call(degp, x, w):
    return pl.pallas_call(
        _mm_scale_body,
        grid=(_N // _BN,),
        in_specs=[
            pl.BlockSpec((_BN, _NT), lambda i: (i, 0)),
            pl.BlockSpec((_BN, _D), lambda i: (i, 0)),
            pl.BlockSpec((_D, _D), lambda i: (0, 0)),
        ],
        out_specs=[
            pl.BlockSpec((_BN, _D), lambda i: (i, 0)),
            pl.BlockSpec((_BN, 1), lambda i: (i, 0)),
        ],
        out_shape=[
            jax.ShapeDtypeStruct((_N, _D), jnp.float32),
            jax.ShapeDtypeStruct((_N, 1), jnp.float32),
        ],
    )(degp, x, w)


def _mid_body(q_ref, hp_ref, dis_ref, b1_ref, w2_ref, out_ref):
    dis = dis_ref[...]
    t = (q_ref[0] + q_ref[1] + hp_ref[...]) * dis + b1_ref[...]
    t = jnp.maximum(t, 0.0)
    out_ref[...] = jnp.dot(t, w2_ref[...],
                           preferred_element_type=jnp.float32) * dis


def _mid_call(q, hp, dis, b1, w2):
    return pl.pallas_call(
        _mid_body,
        grid=(_N // _BN,),
        in_specs=[
            pl.BlockSpec((_NC, _BN, _D), lambda i: (0, i, 0)),
            pl.BlockSpec((_BN, _D), lambda i: (i, 0)),
            pl.BlockSpec((_BN, 1), lambda i: (i, 0)),
            pl.BlockSpec((1, _D), lambda i: (0, 0)),
            pl.BlockSpec((_D, _D), lambda i: (0, 0)),
        ],
        out_specs=pl.BlockSpec((_BN, _D), lambda i: (i, 0)),
        out_shape=jax.ShapeDtypeStruct((_N, _D), jnp.float32),
    )(q, hp, dis, b1, w2)


def _fin_body(q_ref, hp_ref, dis_ref, b2_ref, out_ref):
    out_ref[...] = ((q_ref[0] + q_ref[1] + hp_ref[...]) * dis_ref[...]
                    + b2_ref[...])


def _fin_call(q, hp, dis, b2):
    return pl.pallas_call(
        _fin_body,
        grid=(_N // _BN,),
        in_specs=[
            pl.BlockSpec((_NC, _BN, _D), lambda i: (0, i, 0)),
            pl.BlockSpec((_BN, _D), lambda i: (i, 0)),
            pl.BlockSpec((_BN, 1), lambda i: (i, 0)),
            pl.BlockSpec((1, _D), lambda i: (0, 0)),
        ],
        out_specs=pl.BlockSpec((_BN, _D), lambda i: (i, 0)),
        out_shape=jax.ShapeDtypeStruct((_N, _D), jnp.float32),
    )(q, hp, dis, b2)


def kernel(x, edge_index, edge_weights, W1, b1, W2, b2):
    src = edge_index[0]
    dst = edge_index[1]
    pad = _EPAD - _E
    pk = jnp.concatenate(
        [src + (dst << _SB), jnp.zeros((pad,), jnp.int32)]
    ).reshape(_NT, _C, _K)
    ew = jnp.concatenate(
        [edge_weights, jnp.zeros((pad,), jnp.float32)]
    ).reshape(_NT, _C, _K)

    degp = _deg_call(pk, ew).reshape(_NT, _NPAD)[:, :_N].T
    h1p, dis = _mm_scale_call(degp, x, W1)
    q1 = _agg_call(h1p, pk, ew)
    h2p = _mid_call(q1, h1p, dis, b1.reshape(1, _D), W2)
    q2 = _agg_call(h2p, pk, ew)
    out = _fin_call(q2, h2p, dis, b2.reshape(1, _D))
    return out


# speedup vs baseline: 26.0776x; 2.6350x over previous
import jax
import jax.numpy as jnp
from jax import lax
from jax.experimental import pallas as pl
from jax.experimental.pallas import tpu as pltpu
from jax.experimental.pallas import tpu_sc as plsc

_N = 10000
_E = 320000
_D = 128

_NC = 2
_NS = 16
_NT = _NC * _NS
_K = 128
_C = 80
_EPT = _C * _K
_EPAD = _NT * _EPT
_NPAD = 10240
_RPT = _NPAD // _NS
_SB = 14
_SM = (1 << _SB) - 1


def _sc_mesh():
    return plsc.VectorSubcoreMesh(core_axis_name="c", subcore_axis_name="s")


_SC_PARAMS = pltpu.CompilerParams(needs_layout_passes=False)


def _deg_body(pk_hbm, ew_hbm, deg_out, pk_v, ew_v, deg_v):
    cid = lax.axis_index("c")
    sid = lax.axis_index("s")
    wid = sid * _NC + cid
    pltpu.sync_copy(pk_hbm.at[wid], pk_v)
    pltpu.sync_copy(ew_hbm.at[wid], ew_v)

    def _zero(i, carry):
        for g in range(_D // 16):
            deg_v[i, pl.ds(g * 16, 16)] = jnp.zeros((16,), jnp.float32)
        return carry

    lax.fori_loop(0, _NPAD // _D, _zero, 0)

    def _chunk(j, carry):
        for g in range(_K // 16):
            dst16 = lax.shift_right_logical(pk_v[j, pl.ds(g * 16, 16)], _SB)
            w = ew_v[j, pl.ds(g * 16, 16)]
            plsc.addupdate_scatter(
                deg_v,
                [lax.shift_right_logical(dst16, 7),
                 lax.bitwise_and(dst16, _D - 1)],
                w)
        return carry

    lax.fori_loop(0, _C, _chunk, 0)
    pltpu.sync_copy(deg_v, deg_out.at[wid])


def _deg_call(pk, ew):
    return pl.kernel(
        _deg_body,
        out_type=jax.ShapeDtypeStruct((_NT, _NPAD // _D, _D), jnp.float32),
        mesh=_sc_mesh(),
        scratch_types=[
            pltpu.VMEM((_C, _K), jnp.int32),
            pltpu.VMEM((_C, _K), jnp.float32),
            pltpu.VMEM((_NPAD // _D, _D), jnp.float32),
        ],
        compiler_params=_SC_PARAMS,
    )(pk, ew)


def _agg_body(h_hbm, pk_hbm, ew_hbm, q_out, pk_v, src_ch, dst_ch,
              ew_a, ew_b, rows_a, rows_b, qacc,
              gsem_a, gsem_b, esem_a, esem_b, ssem_a, ssem_b):
    cid = lax.axis_index("c")
    sid = lax.axis_index("s")
    wid = sid * _NC + cid
    pltpu.sync_copy(pk_hbm.at[wid], pk_v)

    def _zb(i, carry):
        for g in range(_D // 16):
            rows_a[i, pl.ds(g * 16, 16)] = jnp.zeros((16,), jnp.float32)
        return carry

    lax.fori_loop(0, _K, _zb, 0)
    for r in range(_RPT // _K):
        pltpu.sync_copy(rows_a, qacc.at[pl.ds(sid * _RPT + r * _K, _K), :])
    plsc.subcore_barrier()

    lanes = lax.iota(jnp.int32, 16)

    def _unpack(j, q):
        for g in range(_K // 16):
            p16 = pk_v[j, pl.ds(g * 16, 16)]
            dst_ch[q, pl.ds(g * 16, 16)] = lax.shift_right_logical(p16, _SB)
            src_ch[q, pl.ds(g * 16, 16)] = lax.bitwise_and(p16, _SM)

    def _fire(j, q, rows, sem):
        _unpack(j, q)
        pltpu.make_async_copy(h_hbm.at[src_ch.at[q]], rows, sem).start()

    def _wait_rows(q, rows, sem):
        pltpu.make_async_copy(h_hbm.at[src_ch.at[q]], rows, sem).wait()

    def _scale(ewbuf, er, rows):
        def _grp(g, c2):
            ew16 = ewbuf[er, pl.ds(g * 16, 16)]
            ridx = g * 16 + lanes

            @plsc.parallel_loop(0, _D, 1, unroll=16)
            def _col(c0):
                cidx = lax.bitwise_and(c0 + lanes, _D - 1)
                v = plsc.load_gather(rows, [ridx, cidx])
                plsc.store_scatter(rows, [ridx, cidx], v * ew16)

            return c2

        lax.fori_loop(0, _K // 16, _grp, 0)

    def _ew_copy(s, buf, sem):
        return pltpu.make_async_copy(ew_hbm.at[wid, pl.ds(s * 8, 8)],
                                     buf, sem)

    _fire(0, 0, rows_a, gsem_a)
    _ew_copy(0, ew_a, esem_a).start()

    def _super_pair(sp, carry):
        for half in (0, 1):
            base = sp * 16 + half * 8
            if half == 0:
                _ew_copy(2 * sp, ew_a, esem_a).wait()
                _ew_copy(2 * sp + 1, ew_b, esem_b).start()
                ewbuf = ew_a
            else:
                _ew_copy(2 * sp + 1, ew_b, esem_b).wait()

                @pl.when(sp < 4)
                def _pf():
                    _ew_copy(2 * sp + 2, ew_a, esem_a).start()

                ewbuf = ew_b

            def _pair(pr, c2, base=base, ewbuf=ewbuf):
                a = base + 2 * pr

                @pl.when(a > 0)
                def _wb():
                    pltpu.make_async_copy(rows_b, qacc.at[dst_ch.at[1]],
                                          ssem_b).wait()

                _fire(a + 1, 1, rows_b, gsem_b)
                _wait_rows(0, rows_a, gsem_a)
                _scale(ewbuf, 2 * pr, rows_a)
                pltpu.async_copy(rows_a, qacc.at[dst_ch.at[0]], ssem_a,
                                 add=True)

                @pl.when(a + 2 < _C)
                def _fn():
                    pltpu.make_async_copy(rows_a, qacc.at[dst_ch.at[0]],
                                          ssem_a).wait()
                    _fire(a + 2, 0, rows_a, gsem_a)

                _wait_rows(1, rows_b, gsem_b)
                _scale(ewbuf, 2 * pr + 1, rows_b)
                pltpu.async_copy(rows_b, qacc.at[dst_ch.at[1]], ssem_b,
                                 add=True)
                return c2

            lax.fori_loop(0, 4, _pair, 0)
        return carry

    lax.fori_loop(0, _C // 16, _super_pair, 0)
    pltpu.make_async_copy(rows_a, qacc.at[dst_ch.at[0]], ssem_a).wait()
    pltpu.make_async_copy(rows_b, qacc.at[dst_ch.at[1]], ssem_b).wait()
    plsc.subcore_barrier()
    @pl.when(sid < _NS - 1)
    def _copy_full():
        pltpu.sync_copy(qacc.at[pl.ds(sid * _RPT, _RPT), :],
                        q_out.at[cid, pl.ds(sid * _RPT, _RPT), :])

    @pl.when(sid == _NS - 1)
    def _copy_tail():
        tail = _N - (_NS - 1) * _RPT
        pltpu.sync_copy(qacc.at[pl.ds((_NS - 1) * _RPT, tail), :],
                        q_out.at[cid, pl.ds((_NS - 1) * _RPT, tail), :])


def _agg_call(h, pk, ew):
    return pl.kernel(
        _agg_body,
        out_type=jax.ShapeDtypeStruct((_NC, _N, _D), jnp.float32),
        mesh=_sc_mesh(),
        scratch_types=[
            pltpu.VMEM((_C, _K), jnp.int32),
            pltpu.VMEM((8, _K), jnp.int32),
            pltpu.VMEM((8, _K), jnp.int32),
            pltpu.VMEM((8, _K), jnp.float32),
            pltpu.VMEM((8, _K), jnp.float32),
            pltpu.VMEM((_K, _D), jnp.float32),
            pltpu.VMEM((_K, _D), jnp.float32),
            pltpu.VMEM_SHARED((_NPAD, _D), jnp.float32),
            pltpu.SemaphoreType.DMA,
            pltpu.SemaphoreType.DMA,
            pltpu.SemaphoreType.DMA,
            pltpu.SemaphoreType.DMA,
            pltpu.SemaphoreType.DMA,
            pltpu.SemaphoreType.DMA,
        ],
        compiler_params=_SC_PARAMS,
    )(h, pk, ew)


_BN = 1000


def _mm_scale_body(degp_ref, x_ref, w_ref, h_ref, dis_ref):
    deg = 1.0 + jnp.sum(degp_ref[...], axis=1)
    dis = lax.rsqrt(deg)[:, None]
    h_ref[...] = jnp.dot(x_ref[...], w_ref[...],
                         preferred_element_type=jnp.float32) * dis
    dis_ref[...] = dis


def _mm_scale_call(degp, x, w):
    return pl.pallas_call(
        _mm_scale_body,
        grid=(_N // _BN,),
        in_specs=[
            pl.BlockSpec((_BN, _NT), lambda i: (i, 0)),
            pl.BlockSpec((_BN, _D), lambda i: (i, 0)),
            pl.BlockSpec((_D, _D), lambda i: (0, 0)),
        ],
        out_specs=[
            pl.BlockSpec((_BN, _D), lambda i: (i, 0)),
            pl.BlockSpec((_BN, 1), lambda i: (i, 0)),
        ],
        out_shape=[
            jax.ShapeDtypeStruct((_N, _D), jnp.float32),
            jax.ShapeDtypeStruct((_N, 1), jnp.float32),
        ],
    )(degp, x, w)


def _mid_body(q_ref, hp_ref, dis_ref, b1_ref, w2_ref, out_ref):
    dis = dis_ref[...]
    t = (q_ref[0] + q_ref[1] + hp_ref[...]) * dis + b1_ref[...]
    t = jnp.maximum(t, 0.0)
    out_ref[...] = jnp.dot(t, w2_ref[...],
                           preferred_element_type=jnp.float32) * dis


def _mid_call(q, hp, dis, b1, w2):
    return pl.pallas_call(
        _mid_body,
        grid=(_N // _BN,),
        in_specs=[
            pl.BlockSpec((_NC, _BN, _D), lambda i: (0, i, 0)),
            pl.BlockSpec((_BN, _D), lambda i: (i, 0)),
            pl.BlockSpec((_BN, 1), lambda i: (i, 0)),
            pl.BlockSpec((1, _D), lambda i: (0, 0)),
            pl.BlockSpec((_D, _D), lambda i: (0, 0)),
        ],
        out_specs=pl.BlockSpec((_BN, _D), lambda i: (i, 0)),
        out_shape=jax.ShapeDtypeStruct((_N, _D), jnp.float32),
    )(q, hp, dis, b1, w2)


def _fin_body(q_ref, hp_ref, dis_ref, b2_ref, out_ref):
    out_ref[...] = ((q_ref[0] + q_ref[1] + hp_ref[...]) * dis_ref[...]
                    + b2_ref[...])


def _fin_call(q, hp, dis, b2):
    return pl.pallas_call(
        _fin_body,
        grid=(_N // _BN,),
        in_specs=[
            pl.BlockSpec((_NC, _BN, _D), lambda i: (0, i, 0)),
            pl.BlockSpec((_BN, _D), lambda i: (i, 0)),
            pl.BlockSpec((_BN, 1), lambda i: (i, 0)),
            pl.BlockSpec((1, _D), lambda i: (0, 0)),
        ],
        out_specs=pl.BlockSpec((_BN, _D), lambda i: (i, 0)),
        out_shape=jax.ShapeDtypeStruct((_N, _D), jnp.float32),
    )(q, hp, dis, b2)


def kernel(x, edge_index, edge_weights, W1, b1, W2, b2):
    src = edge_index[0]
    dst = edge_index[1]
    npad = _EPT - _E // _NT
    pkr = (src + (dst << _SB)).reshape(_NT, _E // _NT)
    ewr = edge_weights.reshape(_NT, _E // _NT)
    dsrc = (jnp.arange(npad, dtype=jnp.int32) * 41) % _N
    ddst = _N + jnp.arange(npad, dtype=jnp.int32)
    dpk = jnp.broadcast_to(dsrc + (ddst << _SB), (_NT, npad))
    pk = jnp.concatenate([pkr, dpk], axis=1).reshape(_NT, _C, _K)
    ew = jnp.concatenate(
        [ewr, jnp.zeros((_NT, npad), jnp.float32)], axis=1
    ).reshape(_NT, _C, _K)

    degp = _deg_call(pk, ew).reshape(_NT, _NPAD)[:, :_N].T
    h1p, dis = _mm_scale_call(degp, x, W1)
    q1 = _agg_call(h1p, pk, ew)
    h2p = _mid_call(q1, h1p, dis, b1.reshape(1, _D), W2)
    q2 = _agg_call(h2p, pk, ew)
    out = _fin_call(q2, h2p, dis, b2.reshape(1, _D))
    return out


# speedup vs baseline: 26.1403x; 1.0024x over previous
import jax
import jax.numpy as jnp
from jax import lax
from jax.experimental import pallas as pl
from jax.experimental.pallas import tpu as pltpu
from jax.experimental.pallas import tpu_sc as plsc

_N = 10000
_E = 320000
_D = 128

_NC = 2
_NS = 16
_NT = _NC * _NS
_K = 128
_C = 80
_EPT = _C * _K
_EPAD = _NT * _EPT
_NPAD = 10240
_RPT = _NPAD // _NS
_SB = 14
_SM = (1 << _SB) - 1


def _sc_mesh():
    return plsc.VectorSubcoreMesh(core_axis_name="c", subcore_axis_name="s")


_SC_PARAMS = pltpu.CompilerParams(needs_layout_passes=False)


def _deg_body(pk_hbm, ew_hbm, deg_out, pk_v, ew_v, deg_v):
    cid = lax.axis_index("c")
    sid = lax.axis_index("s")
    wid = sid * _NC + cid
    pltpu.sync_copy(pk_hbm.at[wid], pk_v)
    pltpu.sync_copy(ew_hbm.at[wid], ew_v)

    def _zero(i, carry):
        for g in range(_D // 16):
            deg_v[i, pl.ds(g * 16, 16)] = jnp.zeros((16,), jnp.float32)
        return carry

    lax.fori_loop(0, _NPAD // _D, _zero, 0)

    def _chunk(j, carry):
        for g in range(_K // 16):
            dst16 = lax.shift_right_logical(pk_v[j, pl.ds(g * 16, 16)], _SB)
            w = ew_v[j, pl.ds(g * 16, 16)]
            plsc.addupdate_scatter(
                deg_v,
                [lax.shift_right_logical(dst16, 7),
                 lax.bitwise_and(dst16, _D - 1)],
                w)
        return carry

    lax.fori_loop(0, _C, _chunk, 0)
    pltpu.sync_copy(deg_v, deg_out.at[wid])


def _deg_call(pk, ew):
    return pl.kernel(
        _deg_body,
        out_type=jax.ShapeDtypeStruct((_NT, _NPAD // _D, _D), jnp.float32),
        mesh=_sc_mesh(),
        scratch_types=[
            pltpu.VMEM((_C, _K), jnp.int32),
            pltpu.VMEM((_C, _K), jnp.float32),
            pltpu.VMEM((_NPAD // _D, _D), jnp.float32),
        ],
        compiler_params=_SC_PARAMS,
    )(pk, ew)


def _agg_body(h_hbm, pk_hbm, ew_hbm, q_out, pk_v, src_ch, dst_ch,
              ew_a, ew_b, rows_a, rows_b, qacc,
              gsem_a1, gsem_a2, gsem_b1, gsem_b2,
              esem_a, esem_b, ssem_a, ssem_b):
    cid = lax.axis_index("c")
    sid = lax.axis_index("s")
    wid = sid * _NC + cid
    pltpu.sync_copy(pk_hbm.at[wid], pk_v)

    def _zb(i, carry):
        for g in range(_D // 16):
            rows_a[i, pl.ds(g * 16, 16)] = jnp.zeros((16,), jnp.float32)
        return carry

    lax.fori_loop(0, _K, _zb, 0)
    for r in range(_RPT // _K):
        pltpu.sync_copy(rows_a, qacc.at[pl.ds(sid * _RPT + r * _K, _K), :])
    plsc.subcore_barrier()

    lanes = lax.iota(jnp.int32, 16)

    def _unpack(j, q):
        for g in range(_K // 16):
            p16 = pk_v[j, pl.ds(g * 16, 16)]
            dst_ch[q, pl.ds(g * 16, 16)] = lax.shift_right_logical(p16, _SB)
            src_ch[q, pl.ds(g * 16, 16)] = lax.bitwise_and(p16, _SM)

    _KH = _K // 2

    def _half_copy(q, lo, rows, sem):
        return pltpu.make_async_copy(
            h_hbm.at[src_ch.at[q, pl.ds(lo, _KH)]],
            rows.at[pl.ds(lo, _KH), :], sem)

    def _fire(j, q, rows, s1, s2):
        _unpack(j, q)
        _half_copy(q, 0, rows, s1).start()
        _half_copy(q, _KH, rows, s2).start()

    def _scale(ewbuf, er, rows, g0, g1):
        def _grp(g, c2):
            ew16 = ewbuf[er, pl.ds(g * 16, 16)]
            ridx = g * 16 + lanes

            @plsc.parallel_loop(0, _D, 1, unroll=16)
            def _col(c0):
                cidx = lax.bitwise_and(c0 + lanes, _D - 1)
                v = plsc.load_gather(rows, [ridx, cidx])
                plsc.store_scatter(rows, [ridx, cidx], v * ew16)

            return c2

        lax.fori_loop(g0, g1, _grp, 0)

    def _ew_copy(s, buf, sem):
        return pltpu.make_async_copy(ew_hbm.at[wid, pl.ds(s * 8, 8)],
                                     buf, sem)

    _fire(0, 0, rows_a, gsem_a1, gsem_a2)
    _ew_copy(0, ew_a, esem_a).start()

    def _super_pair(sp, carry):
        for half in (0, 1):
            base = sp * 16 + half * 8
            if half == 0:
                _ew_copy(2 * sp, ew_a, esem_a).wait()
                _ew_copy(2 * sp + 1, ew_b, esem_b).start()
                ewbuf = ew_a
            else:
                _ew_copy(2 * sp + 1, ew_b, esem_b).wait()

                @pl.when(sp < 4)
                def _pf():
                    _ew_copy(2 * sp + 2, ew_a, esem_a).start()

                ewbuf = ew_b

            def _pair(pr, c2, base=base, ewbuf=ewbuf):
                a = base + 2 * pr

                @pl.when(a > 0)
                def _wb():
                    pltpu.make_async_copy(rows_b, qacc.at[dst_ch.at[1]],
                                          ssem_b).wait()

                _fire(a + 1, 1, rows_b, gsem_b1, gsem_b2)
                _half_copy(0, 0, rows_a, gsem_a1).wait()
                _scale(ewbuf, 2 * pr, rows_a, 0, 4)
                _half_copy(0, _KH, rows_a, gsem_a2).wait()
                _scale(ewbuf, 2 * pr, rows_a, 4, 8)
                pltpu.async_copy(rows_a, qacc.at[dst_ch.at[0]], ssem_a,
                                 add=True)

                @pl.when(a + 2 < _C)
                def _fn():
                    pltpu.make_async_copy(rows_a, qacc.at[dst_ch.at[0]],
                                          ssem_a).wait()
                    _fire(a + 2, 0, rows_a, gsem_a1, gsem_a2)

                _half_copy(1, 0, rows_b, gsem_b1).wait()
                _scale(ewbuf, 2 * pr + 1, rows_b, 0, 4)
                _half_copy(1, _KH, rows_b, gsem_b2).wait()
                _scale(ewbuf, 2 * pr + 1, rows_b, 4, 8)
                pltpu.async_copy(rows_b, qacc.at[dst_ch.at[1]], ssem_b,
                                 add=True)
                return c2

            lax.fori_loop(0, 4, _pair, 0)
        return carry

    lax.fori_loop(0, _C // 16, _super_pair, 0)
    pltpu.make_async_copy(rows_a, qacc.at[dst_ch.at[0]], ssem_a).wait()
    pltpu.make_async_copy(rows_b, qacc.at[dst_ch.at[1]], ssem_b).wait()
    plsc.subcore_barrier()
    @pl.when(sid < _NS - 1)
    def _copy_full():
        pltpu.sync_copy(qacc.at[pl.ds(sid * _RPT, _RPT), :],
                        q_out.at[cid, pl.ds(sid * _RPT, _RPT), :])

    @pl.when(sid == _NS - 1)
    def _copy_tail():
        tail = _N - (_NS - 1) * _RPT
        pltpu.sync_copy(qacc.at[pl.ds((_NS - 1) * _RPT, tail), :],
                        q_out.at[cid, pl.ds((_NS - 1) * _RPT, tail), :])


def _agg_call(h, pk, ew):
    return pl.kernel(
        _agg_body,
        out_type=jax.ShapeDtypeStruct((_NC, _N, _D), jnp.float32),
        mesh=_sc_mesh(),
        scratch_types=[
            pltpu.VMEM((_C, _K), jnp.int32),
            pltpu.VMEM((8, _K), jnp.int32),
            pltpu.VMEM((8, _K), jnp.int32),
            pltpu.VMEM((8, _K), jnp.float32),
            pltpu.VMEM((8, _K), jnp.float32),
            pltpu.VMEM((_K, _D), jnp.float32),
            pltpu.VMEM((_K, _D), jnp.float32),
            pltpu.VMEM_SHARED((_NPAD, _D), jnp.float32),
            pltpu.SemaphoreType.DMA,
            pltpu.SemaphoreType.DMA,
            pltpu.SemaphoreType.DMA,
            pltpu.SemaphoreType.DMA,
            pltpu.SemaphoreType.DMA,
            pltpu.SemaphoreType.DMA,
            pltpu.SemaphoreType.DMA,
            pltpu.SemaphoreType.DMA,
        ],
        compiler_params=_SC_PARAMS,
    )(h, pk, ew)


_BN = 1000


def _mm_scale_body(degp_ref, x_ref, w_ref, h_ref, dis_ref):
    deg = 1.0 + jnp.sum(degp_ref[...], axis=1)
    dis = lax.rsqrt(deg)[:, None]
    h_ref[...] = jnp.dot(x_ref[...], w_ref[...],
                         preferred_element_type=jnp.float32) * dis
    dis_ref[...] = dis


def _mm_scale_call(degp, x, w):
    return pl.pallas_call(
        _mm_scale_body,
        grid=(_N // _BN,),
        in_specs=[
            pl.BlockSpec((_BN, _NT), lambda i: (i, 0)),
            pl.BlockSpec((_BN, _D), lambda i: (i, 0)),
            pl.BlockSpec((_D, _D), lambda i: (0, 0)),
        ],
        out_specs=[
            pl.BlockSpec((_BN, _D), lambda i: (i, 0)),
            pl.BlockSpec((_BN, 1), lambda i: (i, 0)),
        ],
        out_shape=[
            jax.ShapeDtypeStruct((_N, _D), jnp.float32),
            jax.ShapeDtypeStruct((_N, 1), jnp.float32),
        ],
    )(degp, x, w)


def _mid_body(q_ref, hp_ref, dis_ref, b1_ref, w2_ref, out_ref):
    dis = dis_ref[...]
    t = (q_ref[0] + q_ref[1] + hp_ref[...]) * dis + b1_ref[...]
    t = jnp.maximum(t, 0.0)
    out_ref[...] = jnp.dot(t, w2_ref[...],
                           preferred_element_type=jnp.float32) * dis


def _mid_call(q, hp, dis, b1, w2):
    return pl.pallas_call(
        _mid_body,
        grid=(_N // _BN,),
        in_specs=[
            pl.BlockSpec((_NC, _BN, _D), lambda i: (0, i, 0)),
            pl.BlockSpec((_BN, _D), lambda i: (i, 0)),
            pl.BlockSpec((_BN, 1), lambda i: (i, 0)),
            pl.BlockSpec((1, _D), lambda i: (0, 0)),
            pl.BlockSpec((_D, _D), lambda i: (0, 0)),
        ],
        out_specs=pl.BlockSpec((_BN, _D), lambda i: (i, 0)),
        out_shape=jax.ShapeDtypeStruct((_N, _D), jnp.float32),
    )(q, hp, dis, b1, w2)


def _fin_body(q_ref, hp_ref, dis_ref, b2_ref, out_ref):
    out_ref[...] = ((q_ref[0] + q_ref[1] + hp_ref[...]) * dis_ref[...]
                    + b2_ref[...])


def _fin_call(q, hp, dis, b2):
    return pl.pallas_call(
        _fin_body,
        grid=(_N // _BN,),
        in_specs=[
            pl.BlockSpec((_NC, _BN, _D), lambda i: (0, i, 0)),
            pl.BlockSpec((_BN, _D), lambda i: (i, 0)),
            pl.BlockSpec((_BN, 1), lambda i: (i, 0)),
            pl.BlockSpec((1, _D), lambda i: (0, 0)),
        ],
        out_specs=pl.BlockSpec((_BN, _D), lambda i: (i, 0)),
        out_shape=jax.ShapeDtypeStruct((_N, _D), jnp.float32),
    )(q, hp, dis, b2)


def kernel(x, edge_index, edge_weights, W1, b1, W2, b2):
    src = edge_index[0]
    dst = edge_index[1]
    npad = _EPT - _E // _NT
    pkr = (src + (dst << _SB)).reshape(_NT, _E // _NT)
    ewr = edge_weights.reshape(_NT, _E // _NT)
    dsrc = (jnp.arange(npad, dtype=jnp.int32) * 41) % _N
    ddst = _N + jnp.arange(npad, dtype=jnp.int32)
    dpk = jnp.broadcast_to(dsrc + (ddst << _SB), (_NT, npad))
    pk = jnp.concatenate([pkr, dpk], axis=1).reshape(_NT, _C, _K)
    ew = jnp.concatenate(
        [ewr, jnp.zeros((_NT, npad), jnp.float32)], axis=1
    ).reshape(_NT, _C, _K)

    degp = _deg_call(pk, ew).reshape(_NT, _NPAD)[:, :_N].T
    h1p, dis = _mm_scale_call(degp, x, W1)
    q1 = _agg_call(h1p, pk, ew)
    h2p = _mid_call(q1, h1p, dis, b1.reshape(1, _D), W2)
    q2 = _agg_call(h2p, pk, ew)
    out = _fin_call(q2, h2p, dis, b2.reshape(1, _D))
    return out


# speedup vs baseline: 26.2246x; 1.0032x over previous
import jax
import jax.numpy as jnp
from jax import lax
from jax.experimental import pallas as pl
from jax.experimental.pallas import tpu as pltpu
from jax.experimental.pallas import tpu_sc as plsc

_N = 10000
_E = 320000
_D = 128

_NC = 2
_NS = 16
_NT = _NC * _NS
_K = 128
_C = 80
_EPT = _C * _K
_EPAD = _NT * _EPT
_NPAD = 10240
_RPT = _NPAD // _NS
_SB = 14
_SM = (1 << _SB) - 1


def _sc_mesh():
    return plsc.VectorSubcoreMesh(core_axis_name="c", subcore_axis_name="s")


_SC_PARAMS = pltpu.CompilerParams(needs_layout_passes=False)


def _deg_body(pk_hbm, ew_hbm, deg_out, pk_v, ew_v, deg_v):
    cid = lax.axis_index("c")
    sid = lax.axis_index("s")
    wid = sid * _NC + cid
    pltpu.sync_copy(pk_hbm.at[wid], pk_v)
    pltpu.sync_copy(ew_hbm.at[wid], ew_v)

    def _zero(i, carry):
        for g in range(_D // 16):
            deg_v[i, pl.ds(g * 16, 16)] = jnp.zeros((16,), jnp.float32)
        return carry

    lax.fori_loop(0, _NPAD // _D, _zero, 0)

    def _chunk(j, carry):
        for g in range(_K // 16):
            dst16 = lax.shift_right_logical(pk_v[j, pl.ds(g * 16, 16)], _SB)
            w = ew_v[j, pl.ds(g * 16, 16)]
            plsc.addupdate_scatter(
                deg_v,
                [lax.shift_right_logical(dst16, 7),
                 lax.bitwise_and(dst16, _D - 1)],
                w)
        return carry

    lax.fori_loop(0, _C, _chunk, 0)
    pltpu.sync_copy(deg_v, deg_out.at[wid])


def _deg_call(pk, ew):
    return pl.kernel(
        _deg_body,
        out_type=jax.ShapeDtypeStruct((_NT, _NPAD // _D, _D), jnp.float32),
        mesh=_sc_mesh(),
        scratch_types=[
            pltpu.VMEM((_C, _K), jnp.int32),
            pltpu.VMEM((_C, _K), jnp.float32),
            pltpu.VMEM((_NPAD // _D, _D), jnp.float32),
        ],
        compiler_params=_SC_PARAMS,
    )(pk, ew)


def _agg_body(h_hbm, pk_hbm, ew_hbm, q_out, pk_v, src_ch, dst_ch,
              ew_a, ew_b, rows_a, rows_b, qacc,
              gsem_a1, gsem_a2, gsem_b1, gsem_b2,
              esem_a, esem_b, ssem_a, ssem_b):
    cid = lax.axis_index("c")
    sid = lax.axis_index("s")
    wid = sid * _NC + cid
    pltpu.sync_copy(pk_hbm.at[wid], pk_v)

    def _zb(i, carry):
        for g in range(_D // 16):
            rows_a[i, pl.ds(g * 16, 16)] = jnp.zeros((16,), jnp.float32)
        return carry

    lax.fori_loop(0, _K, _zb, 0)
    for r in range(_RPT // _K):
        pltpu.sync_copy(rows_a, qacc.at[pl.ds(sid * _RPT + r * _K, _K), :])
    plsc.subcore_barrier()

    lanes = lax.iota(jnp.int32, 16)

    def _unpack(j, q):
        for g in range(_K // 16):
            p16 = pk_v[j, pl.ds(g * 16, 16)]
            dst_ch[q, pl.ds(g * 16, 16)] = lax.shift_right_logical(p16, _SB)
            src_ch[q, pl.ds(g * 16, 16)] = lax.bitwise_and(p16, _SM)

    _KH = _K // 2

    def _half_copy(q, lo, rows, sem):
        return pltpu.make_async_copy(
            h_hbm.at[src_ch.at[q, pl.ds(lo, _KH)]],
            rows.at[pl.ds(lo, _KH), :], sem)

    def _fire(j, q, rows, s1, s2):
        _unpack(j, q)
        _half_copy(q, 0, rows, s1).start()
        _half_copy(q, _KH, rows, s2).start()

    def _scale(ewbuf, er, rows, g0, g1):
        def _grp(g, c2):
            ew16 = ewbuf[er, pl.ds(g * 16, 16)]
            ridx = g * 16 + lanes

            @plsc.parallel_loop(0, _D, 1, unroll=16)
            def _col(c0):
                cidx = lax.bitwise_and(c0 + lanes, _D - 1)
                v = plsc.load_gather(rows, [ridx, cidx])
                plsc.store_scatter(rows, [ridx, cidx], v * ew16)

            return c2

        lax.fori_loop(g0, g1, _grp, 0)

    def _ew_copy(s, buf, sem):
        return pltpu.make_async_copy(ew_hbm.at[wid, pl.ds(s * 8, 8)],
                                     buf, sem)

    _fire(0, 0, rows_a, gsem_a1, gsem_a2)
    _ew_copy(0, ew_a, esem_a).start()

    def _super_pair(sp, carry):
        for half in (0, 1):
            base = sp * 16 + half * 8
            if half == 0:
                _ew_copy(2 * sp, ew_a, esem_a).wait()
                _ew_copy(2 * sp + 1, ew_b, esem_b).start()
                ewbuf = ew_a
            else:
                _ew_copy(2 * sp + 1, ew_b, esem_b).wait()

                @pl.when(sp < 4)
                def _pf():
                    _ew_copy(2 * sp + 2, ew_a, esem_a).start()

                ewbuf = ew_b

            def _pair(pr, c2, base=base, ewbuf=ewbuf):
                a = base + 2 * pr

                @pl.when(a > 0)
                def _wb():
                    pltpu.make_async_copy(rows_b, qacc.at[dst_ch.at[1]],
                                          ssem_b).wait()

                _fire(a + 1, 1, rows_b, gsem_b1, gsem_b2)
                _half_copy(0, 0, rows_a, gsem_a1).wait()
                _scale(ewbuf, 2 * pr, rows_a, 0, 4)
                _half_copy(0, _KH, rows_a, gsem_a2).wait()
                _scale(ewbuf, 2 * pr, rows_a, 4, 8)
                pltpu.async_copy(rows_a, qacc.at[dst_ch.at[0]], ssem_a,
                                 add=True)

                @pl.when(a + 2 < _C)
                def _fn():
                    pltpu.make_async_copy(rows_a, qacc.at[dst_ch.at[0]],
                                          ssem_a).wait()
                    _fire(a + 2, 0, rows_a, gsem_a1, gsem_a2)

                _half_copy(1, 0, rows_b, gsem_b1).wait()
                _scale(ewbuf, 2 * pr + 1, rows_b, 0, 4)
                _half_copy(1, _KH, rows_b, gsem_b2).wait()
                _scale(ewbuf, 2 * pr + 1, rows_b, 4, 8)
                pltpu.async_copy(rows_b, qacc.at[dst_ch.at[1]], ssem_b,
                                 add=True)
                return c2

            lax.fori_loop(0, 4, _pair, 0)
        return carry

    lax.fori_loop(0, _C // 16, _super_pair, 0)
    pltpu.make_async_copy(rows_a, qacc.at[dst_ch.at[0]], ssem_a).wait()
    pltpu.make_async_copy(rows_b, qacc.at[dst_ch.at[1]], ssem_b).wait()
    plsc.subcore_barrier()
    @pl.when(sid < _NS - 1)
    def _copy_full():
        pltpu.sync_copy(qacc.at[pl.ds(sid * _RPT, _RPT), :],
                        q_out.at[cid, pl.ds(sid * _RPT, _RPT), :])

    @pl.when(sid == _NS - 1)
    def _copy_tail():
        tail = _N - (_NS - 1) * _RPT
        pltpu.sync_copy(qacc.at[pl.ds((_NS - 1) * _RPT, tail), :],
                        q_out.at[cid, pl.ds((_NS - 1) * _RPT, tail), :])


def _agg_call(h, pk, ew):
    return pl.kernel(
        _agg_body,
        out_type=jax.ShapeDtypeStruct((_NC, _N, _D), jnp.float32),
        mesh=_sc_mesh(),
        scratch_types=[
            pltpu.VMEM((_C, _K), jnp.int32),
            pltpu.VMEM((8, _K), jnp.int32),
            pltpu.VMEM((8, _K), jnp.int32),
            pltpu.VMEM((8, _K), jnp.float32),
            pltpu.VMEM((8, _K), jnp.float32),
            pltpu.VMEM((_K, _D), jnp.float32),
            pltpu.VMEM((_K, _D), jnp.float32),
            pltpu.VMEM_SHARED((_NPAD, _D), jnp.float32),
            pltpu.SemaphoreType.DMA,
            pltpu.SemaphoreType.DMA,
            pltpu.SemaphoreType.DMA,
            pltpu.SemaphoreType.DMA,
            pltpu.SemaphoreType.DMA,
            pltpu.SemaphoreType.DMA,
            pltpu.SemaphoreType.DMA,
            pltpu.SemaphoreType.DMA,
        ],
        compiler_params=_SC_PARAMS,
    )(h, pk, ew)


_BN = 1000


def _mm_body(x_ref, w_ref, h_ref):
    h_ref[...] = jnp.dot(x_ref[...], w_ref[...],
                         preferred_element_type=jnp.float32)


def _mm_call(x, w):
    return pl.pallas_call(
        _mm_body,
        grid=(_N // _BN,),
        in_specs=[
            pl.BlockSpec((_BN, _D), lambda i: (i, 0)),
            pl.BlockSpec((_D, _D), lambda i: (0, 0)),
        ],
        out_specs=pl.BlockSpec((_BN, _D), lambda i: (i, 0)),
        out_shape=jax.ShapeDtypeStruct((_N, _D), jnp.float32),
    )(x, w)


def _mm_scale_body(degp_ref, h_ref, ho_ref, dis_ref):
    deg = 1.0 + jnp.sum(degp_ref[...], axis=1)
    dis = lax.rsqrt(deg)[:, None]
    ho_ref[...] = h_ref[...] * dis
    dis_ref[...] = dis


def _mm_scale_call(degp, h):
    return pl.pallas_call(
        _mm_scale_body,
        grid=(_N // _BN,),
        in_specs=[
            pl.BlockSpec((_BN, _NT), lambda i: (i, 0)),
            pl.BlockSpec((_BN, _D), lambda i: (i, 0)),
        ],
        out_specs=[
            pl.BlockSpec((_BN, _D), lambda i: (i, 0)),
            pl.BlockSpec((_BN, 1), lambda i: (i, 0)),
        ],
        out_shape=[
            jax.ShapeDtypeStruct((_N, _D), jnp.float32),
            jax.ShapeDtypeStruct((_N, 1), jnp.float32),
        ],
    )(degp, h)


def _mid_body(q_ref, hp_ref, dis_ref, b1_ref, w2_ref, out_ref):
    dis = dis_ref[...]
    t = (q_ref[0] + q_ref[1] + hp_ref[...]) * dis + b1_ref[...]
    t = jnp.maximum(t, 0.0)
    out_ref[...] = jnp.dot(t, w2_ref[...],
                           preferred_element_type=jnp.float32) * dis


def _mid_call(q, hp, dis, b1, w2):
    return pl.pallas_call(
        _mid_body,
        grid=(_N // _BN,),
        in_specs=[
            pl.BlockSpec((_NC, _BN, _D), lambda i: (0, i, 0)),
            pl.BlockSpec((_BN, _D), lambda i: (i, 0)),
            pl.BlockSpec((_BN, 1), lambda i: (i, 0)),
            pl.BlockSpec((1, _D), lambda i: (0, 0)),
            pl.BlockSpec((_D, _D), lambda i: (0, 0)),
        ],
        out_specs=pl.BlockSpec((_BN, _D), lambda i: (i, 0)),
        out_shape=jax.ShapeDtypeStruct((_N, _D), jnp.float32),
    )(q, hp, dis, b1, w2)


def _fin_body(q_ref, hp_ref, dis_ref, b2_ref, out_ref):
    out_ref[...] = ((q_ref[0] + q_ref[1] + hp_ref[...]) * dis_ref[...]
                    + b2_ref[...])


def _fin_call(q, hp, dis, b2):
    return pl.pallas_call(
        _fin_body,
        grid=(_N // _BN,),
        in_specs=[
            pl.BlockSpec((_NC, _BN, _D), lambda i: (0, i, 0)),
            pl.BlockSpec((_BN, _D), lambda i: (i, 0)),
            pl.BlockSpec((_BN, 1), lambda i: (i, 0)),
            pl.BlockSpec((1, _D), lambda i: (0, 0)),
        ],
        out_specs=pl.BlockSpec((_BN, _D), lambda i: (i, 0)),
        out_shape=jax.ShapeDtypeStruct((_N, _D), jnp.float32),
    )(q, hp, dis, b2)


def kernel(x, edge_index, edge_weights, W1, b1, W2, b2):
    src = edge_index[0]
    dst = edge_index[1]
    npad = _EPT - _E // _NT
    pkr = (src + (dst << _SB)).reshape(_NT, _E // _NT)
    ewr = edge_weights.reshape(_NT, _E // _NT)
    dsrc = (jnp.arange(npad, dtype=jnp.int32) * 41) % _N
    ddst = _N + jnp.arange(npad, dtype=jnp.int32)
    dpk = jnp.broadcast_to(dsrc + (ddst << _SB), (_NT, npad))
    pk = jnp.concatenate([pkr, dpk], axis=1).reshape(_NT, _C, _K)
    ew = jnp.concatenate(
        [ewr, jnp.zeros((_NT, npad), jnp.float32)], axis=1
    ).reshape(_NT, _C, _K)

    h1r = _mm_call(x, W1)
    degp = _deg_call(pk, ew).reshape(_NT, _NPAD)[:, :_N].T
    h1p, dis = _mm_scale_call(degp, h1r)
    q1 = _agg_call(h1p, pk, ew)
    h2p = _mid_call(q1, h1p, dis, b1.reshape(1, _D), W2)
    q2 = _agg_call(h2p, pk, ew)
    out = _fin_call(q2, h2p, dis, b2.reshape(1, _D))
    return out
